# initial kernel scaffold (unmeasured)
import jax
import jax.numpy as jnp
from jax import lax
from jax.experimental import pallas as pl
from jax.experimental.pallas import tpu as pltpu

N_DEV = 4


def kernel(x, w_mat):
    m_per, k = x.shape
    _, n = w_mat.shape
    n_per = n // N_DEV
    n_chunk = 256
    n_chunks = n // n_chunk
    halves = n_per // n_chunk

    def body(x_hbm, w_hbm, out_ref,
             xf_ref, xbf_ref, wf_ref, send_ref, recv_ref,
             xdma_sem, wdma_sems, send_sems, recv_sems):
        me = lax.axis_index("i")

        x_dma = pltpu.make_async_copy(x_hbm, xf_ref, xdma_sem)
        x_dma.start()

        def w_dma(c, buf):
            return pltpu.make_async_copy(
                w_hbm.at[:, pl.ds(c * n_chunk, n_chunk)],
                wf_ref.at[buf],
                wdma_sems.at[buf],
            )

        w_dma(0, 0).start()

        barrier_sem = pltpu.get_barrier_semaphore()
        for d in range(N_DEV):
            @pl.when(me != d)
            def _():
                pl.semaphore_signal(
                    barrier_sem, inc=1,
                    device_id=(d,), device_id_type=pl.DeviceIdType.MESH,
                )
        pl.semaphore_wait(barrier_sem, N_DEV - 1)

        x_dma.wait()
        xbf_ref[...] = xf_ref[...].astype(jnp.bfloat16)

        for c in range(n_chunks):
            buf = c % 2
            if c + 1 < n_chunks:
                w_dma(c + 1, (c + 1) % 2).start()
            w_dma(c, buf).wait()

            wbf = wf_ref[buf].astype(jnp.bfloat16)
            y = jnp.dot(xbf_ref[...], wbf,
                        preferred_element_type=jnp.float32)
            y = jnp.maximum(y, 0.0)

            j, half = c // halves, c % halves
            col = half * n_chunk

            @pl.when(j == me)
            def _():
                out_ref[pl.ds(me * m_per, m_per), col:col + n_chunk] = y

            @pl.when(j != me)
            def _():
                send_ref[j, :, col:col + n_chunk] = y.astype(jnp.bfloat16)

            if half == halves - 1 and True:
                @pl.when(j != me)
                def _():
                    rdma = pltpu.make_async_remote_copy(
                        src_ref=send_ref.at[j],
                        dst_ref=recv_ref.at[me],
                        send_sem=send_sems.at[j],
                        recv_sem=recv_sems.at[me],
                        device_id=(j,),
                        device_id_type=pl.DeviceIdType.MESH,
                    )
                    rdma.start()

        for p in range(N_DEV):
            @pl.when(p != me)
            def _():
                recv = pltpu.make_async_remote_copy(
                    src_ref=send_ref.at[p],
                    dst_ref=recv_ref.at[p],
                    send_sem=send_sems.at[p],
                    recv_sem=recv_sems.at[p],
                    device_id=(p,),
                    device_id_type=pl.DeviceIdType.MESH,
                )
                recv.wait_recv()
                out_ref[pl.ds(p * m_per, m_per), :] = (
                    recv_ref[p].astype(jnp.float32))

        for j in range(N_DEV):
            @pl.when(j != me)
            def _():
                send = pltpu.make_async_remote_copy(
                    src_ref=send_ref.at[j],
                    dst_ref=recv_ref.at[j],
                    send_sem=send_sems.at[j],
                    recv_sem=recv_sems.at[j],
                    device_id=(j,),
                    device_id_type=pl.DeviceIdType.MESH,
                )
                send.wait_send()

    return pl.pallas_call(
        body,
        out_shape=jax.ShapeDtypeStruct((N_DEV * m_per, n_per), jnp.float32),
        in_specs=[
            pl.BlockSpec(memory_space=pltpu.ANY),
            pl.BlockSpec(memory_space=pltpu.ANY),
        ],
        out_specs=pl.BlockSpec(memory_space=pltpu.VMEM),
        scratch_shapes=[
            pltpu.VMEM((m_per, k), jnp.float32),
            pltpu.VMEM((m_per, k), jnp.bfloat16),
            pltpu.VMEM((2, k, n_chunk), jnp.float32),
            pltpu.VMEM((N_DEV, m_per, n_per), jnp.bfloat16),
            pltpu.VMEM((N_DEV, m_per, n_per), jnp.bfloat16),
            pltpu.SemaphoreType.DMA,
            pltpu.SemaphoreType.DMA((2,)),
            pltpu.SemaphoreType.DMA((N_DEV,)),
            pltpu.SemaphoreType.DMA((N_DEV,)),
        ],
        compiler_params=pltpu.CompilerParams(collective_id=0),
    )(x, w_mat)


# baseline (device time: 79113 ns/iter reference)
import jax
import jax.numpy as jnp
from jax import lax
from jax.experimental import pallas as pl
from jax.experimental.pallas import tpu as pltpu

N_DEV = 4


def kernel(x, w_mat):
    m_per, k = x.shape
    _, n = w_mat.shape
    n_per = n // N_DEV
    n_chunk = 256
    n_chunks = n // n_chunk
    halves = n_per // n_chunk

    def body(x_hbm, w_hbm, out_ref,
             xf_ref, xbf_ref, wf_ref, send_ref, recv_ref,
             xdma_sem, wdma_sems, send_sems, recv_sems):
        me = lax.axis_index("i")

        x_dma = pltpu.make_async_copy(x_hbm, xf_ref, xdma_sem)
        x_dma.start()

        def w_dma(c, buf):
            return pltpu.make_async_copy(
                w_hbm.at[:, pl.ds(c * n_chunk, n_chunk)],
                wf_ref.at[buf],
                wdma_sems.at[buf],
            )

        w_dma(0, 0).start()

        barrier_sem = pltpu.get_barrier_semaphore()
        for d in range(N_DEV):
            @pl.when(me != d)
            def _():
                pl.semaphore_signal(
                    barrier_sem, inc=1,
                    device_id=(d,), device_id_type=pl.DeviceIdType.MESH,
                )
        pl.semaphore_wait(barrier_sem, N_DEV - 1)

        x_dma.wait()
        xbf_ref[...] = xf_ref[...].astype(jnp.bfloat16)

        for c in range(n_chunks):
            buf = c % 2
            if c + 1 < n_chunks:
                w_dma(c + 1, (c + 1) % 2).start()
            w_dma(c, buf).wait()

            wbf = wf_ref[buf].astype(jnp.bfloat16)
            y = jnp.dot(xbf_ref[...], wbf,
                        preferred_element_type=jnp.float32)
            y = jnp.maximum(y, 0.0)

            j, half = c // halves, c % halves
            col = half * n_chunk

            @pl.when(j == me)
            def _():
                out_ref[pl.ds(me * m_per, m_per), col:col + n_chunk] = y

            @pl.when(j != me)
            def _():
                send_ref[j, :, col:col + n_chunk] = y.astype(jnp.bfloat16)

            if half == halves - 1 and True:
                @pl.when(j != me)
                def _():
                    rdma = pltpu.make_async_remote_copy(
                        src_ref=send_ref.at[j],
                        dst_ref=recv_ref.at[me],
                        send_sem=send_sems.at[j],
                        recv_sem=recv_sems.at[me],
                        device_id=(j,),
                        device_id_type=pl.DeviceIdType.MESH,
                    )
                    rdma.start()

        for p in range(N_DEV):
            @pl.when(p != me)
            def _():
                recv = pltpu.make_async_remote_copy(
                    src_ref=send_ref.at[p],
                    dst_ref=recv_ref.at[p],
                    send_sem=send_sems.at[p],
                    recv_sem=recv_sems.at[p],
                    device_id=(p,),
                    device_id_type=pl.DeviceIdType.MESH,
                )
                recv.wait_recv()
                out_ref[pl.ds(p * m_per, m_per), :] = (
                    recv_ref[p].astype(jnp.float32))

        for j in range(N_DEV):
            @pl.when(j != me)
            def _():
                send = pltpu.make_async_remote_copy(
                    src_ref=send_ref.at[j],
                    dst_ref=recv_ref.at[j],
                    send_sem=send_sems.at[j],
                    recv_sem=recv_sems.at[j],
                    device_id=(j,),
                    device_id_type=pl.DeviceIdType.MESH,
                )
                send.wait_send()

    return pl.pallas_call(
        body,
        out_shape=jax.ShapeDtypeStruct((N_DEV * m_per, n_per), jnp.float32),
        in_specs=[
            pl.BlockSpec(memory_space=pl.ANY),
            pl.BlockSpec(memory_space=pl.ANY),
        ],
        out_specs=pl.BlockSpec(memory_space=pltpu.VMEM),
        scratch_shapes=[
            pltpu.VMEM((m_per, k), jnp.float32),
            pltpu.VMEM((m_per, k), jnp.bfloat16),
            pltpu.VMEM((2, k, n_chunk), jnp.float32),
            pltpu.VMEM((N_DEV, m_per, n_per), jnp.bfloat16),
            pltpu.VMEM((N_DEV, m_per, n_per), jnp.bfloat16),
            pltpu.SemaphoreType.DMA,
            pltpu.SemaphoreType.DMA((2,)),
            pltpu.SemaphoreType.DMA((N_DEV,)),
            pltpu.SemaphoreType.DMA((N_DEV,)),
        ],
        compiler_params=pltpu.CompilerParams(
            collective_id=0,
            vmem_limit_bytes=60 * 1024 * 1024,
        ),
    )(x, w_mat)


# device time: 55441 ns/iter; 1.4270x vs baseline; 1.4270x over previous
import jax
import jax.numpy as jnp
from jax import lax
from jax.experimental import pallas as pl
from jax.experimental.pallas import tpu as pltpu

N_DEV = 4
SEND_ORDER = (1, 3, 2)


def kernel(x, w_mat):
    m_per, k = x.shape
    _, n = w_mat.shape
    n_per = n // N_DEV
    x_rows = 256
    x_chunks = m_per // x_rows

    def body(x_hbm, w_hbm, out_ref,
             xf_ref, xbf_ref, wf_ref, send_ref, recv_ref,
             xdma_sems, wdma_sems, send_sems, recv_sems):
        me = lax.axis_index("i")

        def x_dma(r, buf):
            return pltpu.make_async_copy(
                x_hbm.at[pl.ds(r * x_rows, x_rows), :],
                xf_ref.at[buf],
                xdma_sems.at[buf],
            )

        def w_dma(j, buf):
            return pltpu.make_async_copy(
                w_hbm.at[:, pl.ds(j * n_per, n_per)],
                wf_ref.at[buf],
                wdma_sems.at[buf],
            )

        x_dma(0, 0).start()
        x_dma(1, 1).start()
        first_j = (me + SEND_ORDER[0]) % N_DEV
        w_dma(first_j, 0).start()

        barrier_sem = pltpu.get_barrier_semaphore()
        for d in range(N_DEV):
            @pl.when(me != d)
            def _():
                pl.semaphore_signal(
                    barrier_sem, inc=1,
                    device_id=(d,), device_id_type=pl.DeviceIdType.MESH,
                )
        pl.semaphore_wait(barrier_sem, N_DEV - 1)

        for r in range(x_chunks):
            buf = r % 2
            x_dma(r, buf).wait()
            if r + 2 < x_chunks:
                x_dma(r + 2, buf).start()
            xbf_ref[pl.ds(r * x_rows, x_rows), :] = (
                xf_ref[buf].astype(jnp.bfloat16))

        block_js = [(me + d) % N_DEV for d in SEND_ORDER] + [me]
        for step in range(N_DEV):
            j = block_js[step]
            buf = step % 2
            w_dma(j, buf).wait()
            if step + 1 < N_DEV:
                w_dma(block_js[step + 1], (step + 1) % 2).start()

            y = jnp.dot(xbf_ref[...], wf_ref[buf].astype(jnp.bfloat16),
                        preferred_element_type=jnp.float32)
            y = jnp.maximum(y, 0.0)

            if step < N_DEV - 1:
                send_ref[j] = y.astype(jnp.bfloat16)
                rdma = pltpu.make_async_remote_copy(
                    src_ref=send_ref.at[j],
                    dst_ref=recv_ref.at[me],
                    send_sem=send_sems.at[j],
                    recv_sem=recv_sems.at[me],
                    device_id=(j,),
                    device_id_type=pl.DeviceIdType.MESH,
                )
                rdma.start()
            else:
                out_ref[pl.ds(me * m_per, m_per), :] = y

        for d in SEND_ORDER:
            p = (me - d) % N_DEV
            recv = pltpu.make_async_remote_copy(
                src_ref=send_ref.at[p],
                dst_ref=recv_ref.at[p],
                send_sem=send_sems.at[p],
                recv_sem=recv_sems.at[p],
                device_id=(p,),
                device_id_type=pl.DeviceIdType.MESH,
            )
            recv.wait_recv()
            out_ref[pl.ds(p * m_per, m_per), :] = (
                recv_ref[p].astype(jnp.float32))

        for d in SEND_ORDER:
            j = (me + d) % N_DEV
            send = pltpu.make_async_remote_copy(
                src_ref=send_ref.at[j],
                dst_ref=recv_ref.at[j],
                send_sem=send_sems.at[j],
                recv_sem=recv_sems.at[j],
                device_id=(j,),
                device_id_type=pl.DeviceIdType.MESH,
            )
            send.wait_send()

    return pl.pallas_call(
        body,
        out_shape=jax.ShapeDtypeStruct((N_DEV * m_per, n_per), jnp.float32),
        in_specs=[
            pl.BlockSpec(memory_space=pl.ANY),
            pl.BlockSpec(memory_space=pl.ANY),
        ],
        out_specs=pl.BlockSpec(memory_space=pltpu.VMEM),
        scratch_shapes=[
            pltpu.VMEM((2, x_rows, k), jnp.float32),
            pltpu.VMEM((m_per, k), jnp.bfloat16),
            pltpu.VMEM((2, k, n_per), jnp.float32),
            pltpu.VMEM((N_DEV, m_per, n_per), jnp.bfloat16),
            pltpu.VMEM((N_DEV, m_per, n_per), jnp.bfloat16),
            pltpu.SemaphoreType.DMA((2,)),
            pltpu.SemaphoreType.DMA((2,)),
            pltpu.SemaphoreType.DMA((N_DEV,)),
            pltpu.SemaphoreType.DMA((N_DEV,)),
        ],
        compiler_params=pltpu.CompilerParams(
            collective_id=0,
            vmem_limit_bytes=60 * 1024 * 1024,
        ),
    )(x, w_mat)
